# baseline (speedup 1.0000x reference)
import functools

import jax
import jax.numpy as jnp
from jax import lax
from jax.experimental import pallas as pl
from jax.experimental.pallas import tpu as pltpu
import jax.experimental.pallas.tpu_sc as plsc

EMBED = 64
B = 4096
L = 20

NC = 2
NS = 16
NW = NC * NS
EX_PER_W = B // NW
CHUNK = 32
N_CHUNK = EX_PER_W // CHUNK
ROWS_PER_CHUNK = CHUNK * L
IDX_COLS = 128
IDX_ROWS = ROWS_PER_CHUNK // IDX_COLS


def _sc_embed_sums_body(mnt_mat, ent_mat, mnt_idx, pos_idx, neg_idx,
                        out_m, out_p, out_n, idx_v, rows_v, acc_v, sem):
    wid = lax.axis_index("s") * NC + lax.axis_index("c")

    for tbl, idx_hbm, out_hbm in ((mnt_mat, mnt_idx, out_m),
                                  (ent_mat, pos_idx, out_p),
                                  (ent_mat, neg_idx, out_n)):
        def chunk_body(cidx, _, tbl=tbl, idx_hbm=idx_hbm, out_hbm=out_hbm):
            ex0 = wid * EX_PER_W + cidx * CHUNK
            pltpu.sync_copy(idx_hbm.at[pl.ds(ex0 * L, ROWS_PER_CHUNK)], idx_v)
            for j in range(IDX_ROWS):
                pltpu.async_copy(tbl.at[idx_v.at[pl.ds(j * IDX_COLS, IDX_COLS)]],
                                 rows_v.at[pl.ds(j * IDX_COLS, IDX_COLS)], sem)
            for j in range(IDX_ROWS):
                pltpu.make_async_copy(tbl.at[idx_v.at[pl.ds(j * IDX_COLS, IDX_COLS)]],
                                      rows_v.at[pl.ds(j * IDX_COLS, IDX_COLS)],
                                      sem).wait()

            def ex_body(e, _):
                base = e * L
                for k in range(EMBED // 16):
                    s = k * 16
                    acc = rows_v[base, pl.ds(s, 16)]
                    for j in range(1, L):
                        acc = acc + rows_v[base + j, pl.ds(s, 16)]
                    acc_v[e, pl.ds(s, 16)] = acc
                return 0

            lax.fori_loop(0, CHUNK, ex_body, 0)
            pltpu.sync_copy(acc_v, out_hbm.at[pl.ds(ex0, CHUNK)])
            return 0

        lax.fori_loop(0, N_CHUNK, chunk_body, 0)


_sc_embed_sums = pl.kernel(
    _sc_embed_sums_body,
    out_type=(jax.ShapeDtypeStruct((B, EMBED), jnp.float32),
              jax.ShapeDtypeStruct((B, EMBED), jnp.float32),
              jax.ShapeDtypeStruct((B, EMBED), jnp.float32)),
    mesh=plsc.VectorSubcoreMesh(core_axis_name="c", subcore_axis_name="s"),
    scratch_types=[
        pltpu.VMEM((ROWS_PER_CHUNK,), jnp.int32),
        pltpu.VMEM((ROWS_PER_CHUNK, EMBED), jnp.float32),
        pltpu.VMEM((CHUNK, EMBED), jnp.float32),
        pltpu.SemaphoreType.DMA,
    ],
    compiler_params=pltpu.CompilerParams(use_tc_tiling_on_sc=False),
)


def _tc_finish_body(m_ref, p_ref, n_ref, mb_ref, eb_ref, sp_ref, sn_ref):
    m = jnp.tanh(m_ref[...] + mb_ref[...])
    p = jnp.tanh(p_ref[...] + eb_ref[...])
    n = jnp.tanh(n_ref[...] + eb_ref[...])
    eps = 1e-12
    rm = lax.rsqrt(jnp.maximum(jnp.sum(m * m, axis=1), eps))
    rp = lax.rsqrt(jnp.maximum(jnp.sum(p * p, axis=1), eps))
    rn = lax.rsqrt(jnp.maximum(jnp.sum(n * n, axis=1), eps))
    mp = jnp.sum(m * p, axis=1)
    mn = jnp.sum(m * n, axis=1)
    sp_ref[...] = -(mp * rm * rp)
    sn_ref[...] = -(mn * rm * rn)


_TC_BLK = 512


def _tc_finish(em, ep, en, mb, eb):
    grid = B // _TC_BLK
    emb_spec = pl.BlockSpec((_TC_BLK, EMBED), lambda i: (i, 0))
    bias_spec = pl.BlockSpec((1, EMBED), lambda i: (0, 0))
    out_spec = pl.BlockSpec((_TC_BLK,), lambda i: (i,))
    return pl.pallas_call(
        _tc_finish_body,
        grid=(grid,),
        in_specs=[emb_spec, emb_spec, emb_spec, bias_spec, bias_spec],
        out_specs=[out_spec, out_spec],
        out_shape=[jax.ShapeDtypeStruct((B,), jnp.float32),
                   jax.ShapeDtypeStruct((B,), jnp.float32)],
    )(em, ep, en, mb, eb)


def kernel(mention_idx, ent_pos_idx, ent_neg_idx, mnt_matrix, ent_matrix,
           mnt_bias, ent_bias):
    mi = mention_idx.astype(jnp.int32).reshape(B * L)
    pi = ent_pos_idx.astype(jnp.int32).reshape(B * L)
    ni = ent_neg_idx.astype(jnp.int32).reshape(B * L)
    em, ep, en = _sc_embed_sums(mnt_matrix, ent_matrix, mi, pi, ni)
    sp, sn = _tc_finish(em, ep, en, mnt_bias.reshape(1, EMBED),
                        ent_bias.reshape(1, EMBED))
    return sp, sn


# speedup vs baseline: 1.0132x; 1.0132x over previous
import functools

import jax
import jax.numpy as jnp
from jax import lax
from jax.experimental import pallas as pl
from jax.experimental.pallas import tpu as pltpu
import jax.experimental.pallas.tpu_sc as plsc

EMBED = 64
B = 4096
L = 20

NC = 2
NS = 16
NW = NC * NS
EX_PER_W = B // NW
CHUNK = 32
N_CHUNK = EX_PER_W // CHUNK
ROWS_PER_CHUNK = CHUNK * L
IDX_COLS = 128
IDX_ROWS = ROWS_PER_CHUNK // IDX_COLS


def _gather_jobs(tbl, jobs, idx_v, rows_v, acc_v, sem):
    wid = lax.axis_index("s") * NC + lax.axis_index("c")

    for idx_hbm, out_hbm in jobs:
        def chunk_body(cidx, _, tbl=tbl, idx_hbm=idx_hbm, out_hbm=out_hbm):
            ex0 = wid * EX_PER_W + cidx * CHUNK
            pltpu.sync_copy(idx_hbm.at[pl.ds(ex0 * L, ROWS_PER_CHUNK)], idx_v)
            for j in range(IDX_ROWS):
                pltpu.async_copy(tbl.at[idx_v.at[pl.ds(j * IDX_COLS, IDX_COLS)]],
                                 rows_v.at[pl.ds(j * IDX_COLS, IDX_COLS)], sem)
            for j in range(IDX_ROWS):
                pltpu.make_async_copy(tbl.at[idx_v.at[pl.ds(j * IDX_COLS, IDX_COLS)]],
                                      rows_v.at[pl.ds(j * IDX_COLS, IDX_COLS)],
                                      sem).wait()

            def ex_body(e, _):
                base = e * L
                for k in range(EMBED // 16):
                    s = k * 16
                    acc = rows_v[base, pl.ds(s, 16)]
                    for j in range(1, L):
                        acc = acc + rows_v[base + j, pl.ds(s, 16)]
                    acc_v[e, pl.ds(s, 16)] = acc
                return 0

            lax.fori_loop(0, CHUNK, ex_body, 0)
            pltpu.sync_copy(acc_v, out_hbm.at[pl.ds(ex0, CHUNK)])
            return 0

        lax.fori_loop(0, N_CHUNK, chunk_body, 0)


def _sc_mnt_body(mnt_mat, mnt_idx, out_m, idx_v, rows_v, acc_v, sem):
    _gather_jobs(mnt_mat, ((mnt_idx, out_m),), idx_v, rows_v, acc_v, sem)


def _sc_ent_body(ent_mat, pos_idx, neg_idx, out_p, out_n,
                 idx_v, rows_v, acc_v, sem):
    _gather_jobs(ent_mat, ((pos_idx, out_p), (neg_idx, out_n)),
                 idx_v, rows_v, acc_v, sem)


_SC_SCRATCH = [
    pltpu.VMEM((ROWS_PER_CHUNK,), jnp.int32),
    pltpu.VMEM((ROWS_PER_CHUNK, EMBED), jnp.float32),
    pltpu.VMEM((CHUNK, EMBED), jnp.float32),
    pltpu.SemaphoreType.DMA,
]

_sc_mnt = pl.kernel(
    _sc_mnt_body,
    out_type=jax.ShapeDtypeStruct((B, EMBED), jnp.float32),
    mesh=plsc.VectorSubcoreMesh(core_axis_name="c", subcore_axis_name="s"),
    scratch_types=_SC_SCRATCH,
    compiler_params=pltpu.CompilerParams(use_tc_tiling_on_sc=False),
)

_sc_ent = pl.kernel(
    _sc_ent_body,
    out_type=(jax.ShapeDtypeStruct((B, EMBED), jnp.float32),
              jax.ShapeDtypeStruct((B, EMBED), jnp.float32)),
    mesh=plsc.VectorSubcoreMesh(core_axis_name="c", subcore_axis_name="s"),
    scratch_types=_SC_SCRATCH,
    compiler_params=pltpu.CompilerParams(use_tc_tiling_on_sc=False),
)


def _tc_finish_body(m_ref, p_ref, n_ref, mb_ref, eb_ref, sp_ref, sn_ref):
    m = jnp.tanh(m_ref[...] + mb_ref[...])
    p = jnp.tanh(p_ref[...] + eb_ref[...])
    n = jnp.tanh(n_ref[...] + eb_ref[...])
    eps = 1e-12
    rm = lax.rsqrt(jnp.maximum(jnp.sum(m * m, axis=1), eps))
    rp = lax.rsqrt(jnp.maximum(jnp.sum(p * p, axis=1), eps))
    rn = lax.rsqrt(jnp.maximum(jnp.sum(n * n, axis=1), eps))
    mp = jnp.sum(m * p, axis=1)
    mn = jnp.sum(m * n, axis=1)
    sp_ref[...] = -(mp * rm * rp)
    sn_ref[...] = -(mn * rm * rn)


_TC_BLK = 512


def _tc_finish(em, ep, en, mb, eb):
    grid = B // _TC_BLK
    emb_spec = pl.BlockSpec((_TC_BLK, EMBED), lambda i: (i, 0))
    bias_spec = pl.BlockSpec((1, EMBED), lambda i: (0, 0))
    out_spec = pl.BlockSpec((_TC_BLK,), lambda i: (i,))
    return pl.pallas_call(
        _tc_finish_body,
        grid=(grid,),
        in_specs=[emb_spec, emb_spec, emb_spec, bias_spec, bias_spec],
        out_specs=[out_spec, out_spec],
        out_shape=[jax.ShapeDtypeStruct((B,), jnp.float32),
                   jax.ShapeDtypeStruct((B,), jnp.float32)],
    )(em, ep, en, mb, eb)


def kernel(mention_idx, ent_pos_idx, ent_neg_idx, mnt_matrix, ent_matrix,
           mnt_bias, ent_bias):
    mi = mention_idx.astype(jnp.int32).reshape(B * L)
    pi = ent_pos_idx.astype(jnp.int32).reshape(B * L)
    ni = ent_neg_idx.astype(jnp.int32).reshape(B * L)
    em = _sc_mnt(mnt_matrix, mi)
    ep, en = _sc_ent(ent_matrix, pi, ni)
    sp, sn = _tc_finish(em, ep, en, mnt_bias.reshape(1, EMBED),
                        ent_bias.reshape(1, EMBED))
    return sp, sn
